# Initial kernel scaffold; baseline (speedup 1.0000x reference)
import jax
import jax.numpy as jnp
from jax.experimental import pallas as pl


def kernel(edge_index, table, W1, att_src1, att_dst1, b1, W2, att_src2, att_dst2, b2):
    raise NotImplementedError("write your pallas kernel here")


# speedup vs baseline: 8.2476x; 8.2476x over previous
import dataclasses
import functools

import jax
import jax.numpy as jnp
from jax import lax
from jax.experimental import pallas as pl
from jax.experimental.pallas import tpu as pltpu
from jax.experimental.pallas import tpu_sc as plsc

N = 50000
D = 64
E = 800000
SLOPE_ATT = 0.2
SLOPE_ACT = 0.01

NC = 2
NS = 16
EP = 800768
PAD = EP - E
EW_A = EP // 32
BLKA = 1088
NBLKA = EW_A // BLKA
EW_C = EP // 16
BLKC = 64
NBLKC = EW_C // BLKC
NHALF = 25000
PACC = 12544
DUMMY = 12512
SLICE_PS = PACC // NS
FCHUNK = 56

_mesh = plsc.VectorSubcoreMesh(
    core_axis_name="c", subcore_axis_name="s", num_cores=NC, num_subcores=NS)


def _sc_params():
    cp = pltpu.CompilerParams()
    if "needs_layout_passes" in pltpu.CompilerParams.__dataclass_fields__:
        cp = dataclasses.replace(cp, needs_layout_passes=False)
    return cp


def _leaky(x, slope):
    return jnp.maximum(x, slope * x)


R = 2000


def _tc_h_common(x, wt_ref, avs_ref, avd_ref, h_ref, as_ref, ad_ref,
                 ms_ref, md_ref, c_ref):
    i = pl.program_id(0)
    h = jnp.dot(x, wt_ref[...], preferred_element_type=jnp.float32)
    a_s = jnp.sum(h * avs_ref[...], axis=1, keepdims=True)
    a_d = jnp.sum(h * avd_ref[...], axis=1, keepdims=True)
    h_ref[...] = jnp.concatenate([h, jnp.zeros((R, D), jnp.float32)], axis=1)
    as_ref[...] = a_s
    ad_ref[...] = a_d
    bs = jnp.max(a_s).reshape(1, 1)
    bd = jnp.max(a_d).reshape(1, 1)

    @pl.when(i == 0)
    def _():
        ms_ref[...] = bs
        md_ref[...] = bd

    @pl.when(i > 0)
    def _():
        ms_ref[...] = jnp.maximum(ms_ref[...], bs)
        md_ref[...] = jnp.maximum(md_ref[...], bd)

    cc = _leaky(ms_ref[...] + md_ref[...], SLOPE_ATT)
    c_ref[...] = jnp.broadcast_to(cc, (1, 16))


def _tc_h1_body(tab_ref, wt_ref, avs_ref, avd_ref,
                h_ref, as_ref, ad_ref, ms_ref, md_ref, c_ref):
    i = pl.program_id(0)
    rows = lax.broadcasted_iota(jnp.int32, (R, 1), 0) + i * R
    x = jnp.where(rows != 0, tab_ref[...], 0.0)
    _tc_h_common(x, wt_ref, avs_ref, avd_ref, h_ref, as_ref, ad_ref,
                 ms_ref, md_ref, c_ref)


def _tc_h2_body(xb_ref, hp_ref, srd_ref, b_ref, wt_ref, avs_ref, avd_ref,
                h_ref, as_ref, ad_ref, ms_ref, md_ref, c_ref):
    x = _leaky(xb_ref[...] + srd_ref[...] * hp_ref[...][:, :D] + b_ref[...],
               SLOPE_ACT)
    _tc_h_common(x, wt_ref, avs_ref, avd_ref, h_ref, as_ref, ad_ref,
                 ms_ref, md_ref, c_ref)


_h_outs = (
    jax.ShapeDtypeStruct((N, 2 * D), jnp.float32),
    jax.ShapeDtypeStruct((N, 1), jnp.float32),
    jax.ShapeDtypeStruct((N, 1), jnp.float32),
    jax.ShapeDtypeStruct((1, 1), jnp.float32),
    jax.ShapeDtypeStruct((1, 1), jnp.float32),
    jax.ShapeDtypeStruct((1, 16), jnp.float32),
)
_h_out_specs = [
    pl.BlockSpec((R, 2 * D), lambda i: (i, 0)),
    pl.BlockSpec((R, 1), lambda i: (i, 0)),
    pl.BlockSpec((R, 1), lambda i: (i, 0)),
    pl.BlockSpec((1, 1), lambda i: (0, 0)),
    pl.BlockSpec((1, 1), lambda i: (0, 0)),
    pl.BlockSpec((1, 16), lambda i: (0, 0)),
]
_w_specs = [
    pl.BlockSpec((D, D), lambda i: (0, 0)),
    pl.BlockSpec((1, D), lambda i: (0, 0)),
    pl.BlockSpec((1, D), lambda i: (0, 0)),
]

_tc_h1 = pl.pallas_call(
    _tc_h1_body,
    grid=(N // R,),
    in_specs=[pl.BlockSpec((R, D), lambda i: (i, 0))] + _w_specs,
    out_specs=_h_out_specs,
    out_shape=_h_outs,
)

_tc_h2 = pl.pallas_call(
    _tc_h2_body,
    grid=(N // R,),
    in_specs=[
        pl.BlockSpec((R, D), lambda i: (i, 0)),
        pl.BlockSpec((R, 2 * D), lambda i: (i, 0)),
        pl.BlockSpec((R, 1), lambda i: (i, 0)),
        pl.BlockSpec((1, D), lambda i: (0, 0)),
    ] + _w_specs,
    out_specs=_h_out_specs,
    out_shape=_h_outs,
)

BN = 10000


NR0, NR1 = 500, 100


def _tc_r_body(p_ref, as_ref, ad_ref, c_ref, rd_ref, srd_ref):
    cval = c_ref[...][0, 0]
    den = jnp.sum(p_ref[...], axis=0)
    ws = jnp.exp(_leaky(as_ref[...] + ad_ref[...], SLOPE_ATT) - cval)
    r = 1.0 / (den + ws + 1e-16)
    rd_ref[...] = r
    srd_ref[...] = ws * r


_tc_r = pl.pallas_call(
    _tc_r_body,
    grid=(1,),
    in_specs=[
        pl.BlockSpec((32, NR0, NR1), lambda i: (0, 0, 0)),
        pl.BlockSpec((NR0, NR1), lambda i: (0, 0)),
        pl.BlockSpec((NR0, NR1), lambda i: (0, 0)),
        pl.BlockSpec((1, 16), lambda i: (0, 0)),
    ],
    out_specs=[
        pl.BlockSpec((NR0, NR1), lambda i: (0, 0)),
        pl.BlockSpec((NR0, NR1), lambda i: (0, 0)),
    ],
    out_shape=(
        jax.ShapeDtypeStruct((NR0, NR1), jnp.float32),
        jax.ShapeDtypeStruct((NR0, NR1), jnp.float32),
    ),
)


def _tc_fin_body(xb_ref, hp_ref, srd_ref, b_ref, o_ref):
    o_ref[...] = _leaky(
        xb_ref[...] + srd_ref[...] * hp_ref[...][:, :D] + b_ref[...],
        SLOPE_ACT)


_tc_fin = pl.pallas_call(
    _tc_fin_body,
    grid=(N // R,),
    in_specs=[
        pl.BlockSpec((R, D), lambda i: (i, 0)),
        pl.BlockSpec((R, 2 * D), lambda i: (i, 0)),
        pl.BlockSpec((R, 1), lambda i: (i, 0)),
        pl.BlockSpec((1, D), lambda i: (0, 0)),
    ],
    out_specs=pl.BlockSpec((R, D), lambda i: (i, 0)),
    out_shape=jax.ShapeDtypeStruct((N, D), jnp.float32),
)


@functools.partial(
    pl.kernel,
    out_type=(
        jax.ShapeDtypeStruct((EP,), jnp.float32),
        jax.ShapeDtypeStruct((32, 1, N), jnp.float32),
    ),
    mesh=_mesh,
    compiler_params=_sc_params(),
    scratch_types=[
        pltpu.VMEM((N,), jnp.float32),
        pltpu.VMEM((N,), jnp.float32),
        pltpu.VMEM((BLKA,), jnp.int32),
        pltpu.VMEM((BLKA,), jnp.int32),
        pltpu.VMEM((BLKA,), jnp.float32),
        pltpu.VMEM((16,), jnp.float32),
        pltpu.VMEM((16,), jnp.int32),
    ],
)
def _sc_a(src_hbm, dst_hbm, asrc_hbm, adst_hbm, c_hbm,
          w_hbm, part_hbm, A, Bv, sblk, dblk, wblk, cb, tk):
    wid = lax.axis_index("s") * NC + lax.axis_index("c")
    base = wid * EW_A
    pltpu.sync_copy(asrc_hbm, A)
    pltpu.sync_copy(adst_hbm, Bv)
    pltpu.sync_copy(c_hbm.at[0], cb)
    c16 = cb[...]
    lane = lax.iota(jnp.int32, 16)

    @pl.loop(0, NBLKA)
    def _(b):
        off = base + b * BLKA
        pltpu.sync_copy(src_hbm.at[pl.ds(off, BLKA)], sblk)
        pltpu.sync_copy(dst_hbm.at[pl.ds(off, BLKA)], dblk)

        @pl.loop(0, BLKA, step=16)
        def _(j):
            si = sblk[pl.ds(j, 16)]
            di = dblk[pl.ds(j, 16)]
            e = plsc.load_gather(A, [si]) + plsc.load_gather(Bv, [di])
            w = jnp.exp(_leaky(e, SLOPE_ATT) - c16)
            eid = off + j + lane
            wblk[pl.ds(j, 16)] = jnp.where(eid < E, w, 0.0)

        pltpu.sync_copy(wblk, w_hbm.at[pl.ds(off, BLKA)])

    @pl.loop(0, N, step=16)
    def _(i):
        A[pl.ds(i, 16)] = jnp.zeros((16,), jnp.float32)

    @pl.loop(0, NBLKA)
    def _(b):
        off = base + b * BLKA
        pltpu.sync_copy(dst_hbm.at[pl.ds(off, BLKA)], dblk)
        pltpu.sync_copy(w_hbm.at[pl.ds(off, BLKA)], wblk)

        @pl.loop(0, BLKA, step=16)
        def _(j):
            di = dblk[pl.ds(j, 16)]
            w = wblk[pl.ds(j, 16)]
            k, v = plsc.sort_key_val(di, w)
            s = plsc.cumsum(v)
            tk[...] = k
            kn = plsc.load_gather(tk, [jnp.minimum(lane + 1, 15)])
            is_last = (k != kn) | (lane == 15)
            plsc.addupdate_scatter(A, [k], s, mask=is_last)
            plsc.addupdate_scatter(A, [kn], -s, mask=is_last & (lane != 15))

    pltpu.sync_copy(A, part_hbm.at[wid, 0])


@functools.partial(
    pl.kernel,
    out_type=jax.ShapeDtypeStruct((EP,), jnp.float32),
    mesh=_mesh,
    compiler_params=_sc_params(),
    scratch_types=[
        pltpu.VMEM((N,), jnp.float32),
        pltpu.VMEM((BLKA,), jnp.int32),
        pltpu.VMEM((BLKA,), jnp.float32),
        pltpu.VMEM((BLKA,), jnp.float32),
    ],
)
def _sc_b(dst_hbm, w_hbm, rden_hbm, alpha_hbm, rden_t, dblk, wblk, ablk):
    wid = lax.axis_index("s") * NC + lax.axis_index("c")
    base = wid * EW_A
    pltpu.sync_copy(rden_hbm, rden_t)

    @pl.loop(0, NBLKA)
    def _(b):
        off = base + b * BLKA
        pltpu.sync_copy(dst_hbm.at[pl.ds(off, BLKA)], dblk)
        pltpu.sync_copy(w_hbm.at[pl.ds(off, BLKA)], wblk)

        @pl.loop(0, BLKA, step=16)
        def _(j):
            di = dblk[pl.ds(j, 16)]
            ablk[pl.ds(j, 16)] = (
                wblk[pl.ds(j, 16)] * plsc.load_gather(rden_t, [di]))

        pltpu.sync_copy(ablk, alpha_hbm.at[pl.ds(off, BLKA)])


@functools.partial(
    pl.kernel,
    out_type=jax.ShapeDtypeStruct((NC, PACC, 2 * D), jnp.float32),
    mesh=_mesh,
    compiler_params=_sc_params(),
    scratch_types=[
        pltpu.VMEM((BLKC,), jnp.int32),
        pltpu.VMEM((BLKC,), jnp.int32),
        pltpu.VMEM((1, BLKC), jnp.int32),
        pltpu.VMEM((BLKC,), jnp.float32),
        pltpu.VMEM((BLKC,), jnp.float32),
        pltpu.VMEM((BLKC, 2 * D), jnp.float32),
        pltpu.VMEM((BLKC, 2 * D), jnp.float32),
        pltpu.VMEM_SHARED((PACC, 2 * D), jnp.float32),
    ],
)
def _sc_c(src_hbm, dst_hbm, alpha_hbm, h_hbm,
          out_hbm, sblk, dblk, ldst, alp, par, rowsg, rows, acc):
    cid = lax.axis_index("c")
    sid = lax.axis_index("s")
    lane = lax.iota(jnp.int32, 16)

    @pl.loop(0, BLKC)
    def _(i):
        @pl.loop(0, 2 * D, step=16)
        def _(q):
            rows[i, pl.ds(q, 16)] = jnp.zeros((16,), jnp.float32)

    zbase = sid * SLICE_PS
    for t in range(SLICE_PS // FCHUNK):
        pltpu.sync_copy(rows.at[pl.ds(0, FCHUNK)],
                        acc.at[pl.ds(zbase + t * FCHUNK, FCHUNK)])
    plsc.subcore_barrier()

    wbase = sid * EW_C

    @pl.loop(0, NBLKC)
    def _(b):
        off = wbase + b * BLKC
        pltpu.sync_copy(src_hbm.at[pl.ds(off, BLKC)], sblk)
        pltpu.sync_copy(dst_hbm.at[pl.ds(off, BLKC)], dblk)
        pltpu.sync_copy(alpha_hbm.at[pl.ds(off, BLKC)], alp)

        @pl.loop(0, BLKC, step=16)
        def _(j):
            d16 = dblk[pl.ds(j, 16)]
            a = alp[pl.ds(j, 16)]
            ld = d16 - cid * NHALF
            inr = (ld >= 0) & (ld < NHALF)
            alp[pl.ds(j, 16)] = jnp.where(inr, a, 0.0)
            par[pl.ds(j, 16)] = (ld & 1).astype(jnp.float32)
            ldst[0, pl.ds(j, 16)] = jnp.where(
                inr, lax.shift_right_arithmetic(ld, 1), DUMMY + lane)

        pltpu.sync_copy(h_hbm.at[sblk], rowsg)

        @pl.loop(0, BLKC)
        def _(i):
            i16 = jnp.full((16,), i, jnp.int32)
            asp = plsc.load_gather(alp, [i16])
            aR = asp * plsc.load_gather(par, [i16])
            aL = asp - aR
            for q in range(D // 16):
                v = rowsg[i, pl.ds(q * 16, 16)]
                rows[i, pl.ds(q * 16, 16)] = v * aL
                rows[i, pl.ds(D + q * 16, 16)] = v * aR

        pltpu.sync_copy(rows, acc.at[ldst.at[0]], add=True)

    plsc.subcore_barrier()

    fbase = sid * SLICE_PS
    for t in range(SLICE_PS // FCHUNK):
        pltpu.sync_copy(acc.at[pl.ds(fbase + t * FCHUNK, FCHUNK)],
                        out_hbm.at[cid, pl.ds(fbase + t * FCHUNK, FCHUNK)])


def _gat_layer(x_args, src_p, dst_p, first):
    if first:
        tab, wt, avs, avd = x_args
        h, a_s, a_d, _, _, c = _tc_h1(tab, wt, avs, avd)
    else:
        xb, hp, srd, b, wt, avs, avd = x_args
        h, a_s, a_d, _, _, c = _tc_h2(xb, hp, srd, b, wt, avs, avd)
    w, parts = _sc_a(src_p, dst_p, a_s.reshape(N), a_d.reshape(N), c)
    rden, srd2 = _tc_r(parts.reshape(32, NR0, NR1), a_s.reshape(NR0, NR1),
                       a_d.reshape(NR0, NR1), c)
    alpha = _sc_b(dst_p, w, rden.reshape(N))
    out_sc = _sc_c(src_p, dst_p, alpha, h).reshape(NC, 2 * PACC, D)
    outcat = jnp.concatenate([out_sc[0, :NHALF], out_sc[1, :NHALF]], axis=0)
    return outcat, h, srd2.reshape(N, 1)


def kernel(edge_index, table, W1, att_src1, att_dst1, b1,
           W2, att_src2, att_dst2, b2):
    padidx = jnp.arange(PAD, dtype=jnp.int32) % 256
    src_p = jnp.concatenate([edge_index[0].astype(jnp.int32), padidx])
    dst_p = jnp.concatenate([edge_index[1].astype(jnp.int32), padidx])

    out1, h1, srd1 = _gat_layer(
        (table, W1.T, att_src1.reshape(1, D), att_dst1.reshape(1, D)),
        src_p, dst_p, first=True)
    out2, h2, srd2 = _gat_layer(
        (out1, h1, srd1, b1.reshape(1, D), W2.T,
         att_src2.reshape(1, D), att_dst2.reshape(1, D)),
        src_p, dst_p, first=False)
    return _tc_fin(out2, h2, srd2, b2.reshape(1, D))


# speedup vs baseline: 20.8776x; 2.5314x over previous
import dataclasses
import functools

import jax
import jax.numpy as jnp
from jax import lax
from jax.experimental import pallas as pl
from jax.experimental.pallas import tpu as pltpu
from jax.experimental.pallas import tpu_sc as plsc

N = 50000
D = 64
E = 800000
SLOPE_ATT = 0.2
SLOPE_ACT = 0.01

NC = 2
NS = 16
EP = 800768
PAD = EP - E
EW_A = EP // 32
BLKA = 1088
NBLKA = EW_A // BLKA
EW_C = EP // 16
BLKC = 64
NBLKC = EW_C // BLKC
NHALF = 25000
PACC = 12544
DUMMY = 12512
SLICE_PS = PACC // NS
FCHUNK = 56

_mesh = plsc.VectorSubcoreMesh(
    core_axis_name="c", subcore_axis_name="s", num_cores=NC, num_subcores=NS)


def _sc_params():
    cp = pltpu.CompilerParams()
    if "needs_layout_passes" in pltpu.CompilerParams.__dataclass_fields__:
        cp = dataclasses.replace(cp, needs_layout_passes=False)
    return cp


def _leaky(x, slope):
    return jnp.maximum(x, slope * x)


R = 2000


def _tc_h_common(x, wt_ref, avs_ref, avd_ref, h_ref, as_ref, ad_ref,
                 ms_ref, md_ref, c_ref):
    i = pl.program_id(0)
    h = jnp.dot(x, wt_ref[...], preferred_element_type=jnp.float32)
    a_s = jnp.sum(h * avs_ref[...], axis=1, keepdims=True)
    a_d = jnp.sum(h * avd_ref[...], axis=1, keepdims=True)
    h_ref[...] = jnp.concatenate([h, jnp.zeros((R, D), jnp.float32)], axis=1)
    as_ref[...] = a_s
    ad_ref[...] = a_d
    bs = jnp.max(a_s).reshape(1, 1)
    bd = jnp.max(a_d).reshape(1, 1)

    @pl.when(i == 0)
    def _():
        ms_ref[...] = bs
        md_ref[...] = bd

    @pl.when(i > 0)
    def _():
        ms_ref[...] = jnp.maximum(ms_ref[...], bs)
        md_ref[...] = jnp.maximum(md_ref[...], bd)

    cc = _leaky(ms_ref[...] + md_ref[...], SLOPE_ATT)
    c_ref[...] = jnp.broadcast_to(cc, (1, 16))


def _tc_h1_body(tab_ref, wt_ref, avs_ref, avd_ref,
                h_ref, as_ref, ad_ref, ms_ref, md_ref, c_ref):
    i = pl.program_id(0)
    rows = lax.broadcasted_iota(jnp.int32, (R, 1), 0) + i * R
    x = jnp.where(rows != 0, tab_ref[...], 0.0)
    _tc_h_common(x, wt_ref, avs_ref, avd_ref, h_ref, as_ref, ad_ref,
                 ms_ref, md_ref, c_ref)


def _tc_h2_body(xb_ref, hp_ref, srd_ref, b_ref, wt_ref, avs_ref, avd_ref,
                h_ref, as_ref, ad_ref, ms_ref, md_ref, c_ref):
    x = _leaky(xb_ref[...] + srd_ref[...] * hp_ref[...][:, :D] + b_ref[...],
               SLOPE_ACT)
    _tc_h_common(x, wt_ref, avs_ref, avd_ref, h_ref, as_ref, ad_ref,
                 ms_ref, md_ref, c_ref)


_h_outs = (
    jax.ShapeDtypeStruct((N, 2 * D), jnp.float32),
    jax.ShapeDtypeStruct((N, 1), jnp.float32),
    jax.ShapeDtypeStruct((N, 1), jnp.float32),
    jax.ShapeDtypeStruct((1, 1), jnp.float32),
    jax.ShapeDtypeStruct((1, 1), jnp.float32),
    jax.ShapeDtypeStruct((1, 16), jnp.float32),
)
_h_out_specs = [
    pl.BlockSpec((R, 2 * D), lambda i: (i, 0)),
    pl.BlockSpec((R, 1), lambda i: (i, 0)),
    pl.BlockSpec((R, 1), lambda i: (i, 0)),
    pl.BlockSpec((1, 1), lambda i: (0, 0)),
    pl.BlockSpec((1, 1), lambda i: (0, 0)),
    pl.BlockSpec((1, 16), lambda i: (0, 0)),
]
_w_specs = [
    pl.BlockSpec((D, D), lambda i: (0, 0)),
    pl.BlockSpec((1, D), lambda i: (0, 0)),
    pl.BlockSpec((1, D), lambda i: (0, 0)),
]

_tc_h1 = pl.pallas_call(
    _tc_h1_body,
    grid=(N // R,),
    in_specs=[pl.BlockSpec((R, D), lambda i: (i, 0))] + _w_specs,
    out_specs=_h_out_specs,
    out_shape=_h_outs,
)

_tc_h2 = pl.pallas_call(
    _tc_h2_body,
    grid=(N // R,),
    in_specs=[
        pl.BlockSpec((R, D), lambda i: (i, 0)),
        pl.BlockSpec((R, 2 * D), lambda i: (i, 0)),
        pl.BlockSpec((R, 1), lambda i: (i, 0)),
        pl.BlockSpec((1, D), lambda i: (0, 0)),
    ] + _w_specs,
    out_specs=_h_out_specs,
    out_shape=_h_outs,
)

BN = 10000


NR0, NR1 = 500, 100


def _tc_r_body(p_ref, as_ref, ad_ref, c_ref, rd_ref, srd_ref):
    cval = c_ref[...][0, 0]
    den = jnp.sum(p_ref[...], axis=0)
    ws = jnp.exp(_leaky(as_ref[...] + ad_ref[...], SLOPE_ATT) - cval)
    r = 1.0 / (den + ws + 1e-16)
    rd_ref[...] = r
    srd_ref[...] = ws * r


_tc_r = pl.pallas_call(
    _tc_r_body,
    grid=(1,),
    in_specs=[
        pl.BlockSpec((32, NR0, NR1), lambda i: (0, 0, 0)),
        pl.BlockSpec((NR0, NR1), lambda i: (0, 0)),
        pl.BlockSpec((NR0, NR1), lambda i: (0, 0)),
        pl.BlockSpec((1, 16), lambda i: (0, 0)),
    ],
    out_specs=[
        pl.BlockSpec((NR0, NR1), lambda i: (0, 0)),
        pl.BlockSpec((NR0, NR1), lambda i: (0, 0)),
    ],
    out_shape=(
        jax.ShapeDtypeStruct((NR0, NR1), jnp.float32),
        jax.ShapeDtypeStruct((NR0, NR1), jnp.float32),
    ),
)


def _tc_fin_body(xb_ref, hp_ref, srd_ref, b_ref, o_ref):
    o_ref[...] = _leaky(
        xb_ref[...] + srd_ref[...] * hp_ref[...][:, :D] + b_ref[...],
        SLOPE_ACT)


_tc_fin = pl.pallas_call(
    _tc_fin_body,
    grid=(N // R,),
    in_specs=[
        pl.BlockSpec((R, D), lambda i: (i, 0)),
        pl.BlockSpec((R, 2 * D), lambda i: (i, 0)),
        pl.BlockSpec((R, 1), lambda i: (i, 0)),
        pl.BlockSpec((1, D), lambda i: (0, 0)),
    ],
    out_specs=pl.BlockSpec((R, D), lambda i: (i, 0)),
    out_shape=jax.ShapeDtypeStruct((N, D), jnp.float32),
)


@functools.partial(
    pl.kernel,
    out_type=(
        jax.ShapeDtypeStruct((EP,), jnp.float32),
        jax.ShapeDtypeStruct((32, 1, N), jnp.float32),
    ),
    mesh=_mesh,
    compiler_params=_sc_params(),
    scratch_types=[
        pltpu.VMEM((N,), jnp.float32),
        pltpu.VMEM((N,), jnp.float32),
        pltpu.VMEM((BLKA,), jnp.int32),
        pltpu.VMEM((BLKA,), jnp.int32),
        pltpu.VMEM((BLKA,), jnp.float32),
        pltpu.VMEM((16,), jnp.float32),
        pltpu.VMEM((16,), jnp.int32),
    ],
)
def _sc_a(src_hbm, dst_hbm, asrc_hbm, adst_hbm, c_hbm,
          w_hbm, part_hbm, A, Bv, sblk, dblk, wblk, cb, tk):
    wid = lax.axis_index("s") * NC + lax.axis_index("c")
    base = wid * EW_A
    pltpu.sync_copy(asrc_hbm, A)
    pltpu.sync_copy(adst_hbm, Bv)
    pltpu.sync_copy(c_hbm.at[0], cb)
    c16 = cb[...]
    lane = lax.iota(jnp.int32, 16)

    @pl.loop(0, NBLKA)
    def _(b):
        off = base + b * BLKA
        pltpu.sync_copy(src_hbm.at[pl.ds(off, BLKA)], sblk)
        pltpu.sync_copy(dst_hbm.at[pl.ds(off, BLKA)], dblk)

        @pl.loop(0, BLKA, step=16)
        def _(j):
            si = sblk[pl.ds(j, 16)]
            di = dblk[pl.ds(j, 16)]
            e = plsc.load_gather(A, [si]) + plsc.load_gather(Bv, [di])
            w = jnp.exp(_leaky(e, SLOPE_ATT) - c16)
            eid = off + j + lane
            wblk[pl.ds(j, 16)] = jnp.where(eid < E, w, 0.0)

        pltpu.sync_copy(wblk, w_hbm.at[pl.ds(off, BLKA)])

    @pl.loop(0, N, step=16)
    def _(i):
        A[pl.ds(i, 16)] = jnp.zeros((16,), jnp.float32)

    @pl.loop(0, NBLKA)
    def _(b):
        off = base + b * BLKA
        pltpu.sync_copy(dst_hbm.at[pl.ds(off, BLKA)], dblk)
        pltpu.sync_copy(w_hbm.at[pl.ds(off, BLKA)], wblk)

        @pl.loop(0, BLKA, step=16)
        def _(j):
            di = dblk[pl.ds(j, 16)]
            w = wblk[pl.ds(j, 16)]
            k, v = plsc.sort_key_val(di, w)
            s = plsc.cumsum(v)
            tk[...] = k
            kn = plsc.load_gather(tk, [jnp.minimum(lane + 1, 15)])
            is_last = (k != kn) | (lane == 15)
            plsc.addupdate_scatter(A, [k], s, mask=is_last)
            plsc.addupdate_scatter(A, [kn], -s, mask=is_last & (lane != 15))

    pltpu.sync_copy(A, part_hbm.at[wid, 0])


@functools.partial(
    pl.kernel,
    out_type=jax.ShapeDtypeStruct((EP,), jnp.float32),
    mesh=_mesh,
    compiler_params=_sc_params(),
    scratch_types=[
        pltpu.VMEM((N,), jnp.float32),
        pltpu.VMEM((BLKA,), jnp.int32),
        pltpu.VMEM((BLKA,), jnp.float32),
        pltpu.VMEM((BLKA,), jnp.float32),
    ],
)
def _sc_b(dst_hbm, w_hbm, rden_hbm, alpha_hbm, rden_t, dblk, wblk, ablk):
    wid = lax.axis_index("s") * NC + lax.axis_index("c")
    base = wid * EW_A
    pltpu.sync_copy(rden_hbm, rden_t)

    @pl.loop(0, NBLKA)
    def _(b):
        off = base + b * BLKA
        pltpu.sync_copy(dst_hbm.at[pl.ds(off, BLKA)], dblk)
        pltpu.sync_copy(w_hbm.at[pl.ds(off, BLKA)], wblk)

        @pl.loop(0, BLKA, step=16)
        def _(j):
            di = dblk[pl.ds(j, 16)]
            ablk[pl.ds(j, 16)] = (
                wblk[pl.ds(j, 16)] * plsc.load_gather(rden_t, [di]))

        pltpu.sync_copy(ablk, alpha_hbm.at[pl.ds(off, BLKA)])


MCH = 1024
T_PAIRS = NBLKC // 2


@functools.partial(
    pl.kernel,
    out_type=jax.ShapeDtypeStruct((NC, PACC, 2 * D), jnp.float32),
    mesh=_mesh,
    compiler_params=_sc_params(),
    scratch_types=[
        pltpu.VMEM((MCH,), jnp.int32),
        pltpu.VMEM((MCH,), jnp.int32),
        pltpu.VMEM((MCH,), jnp.float32),
        pltpu.VMEM((BLKC,), jnp.float32),
        pltpu.VMEM((BLKC,), jnp.float32),
        pltpu.VMEM((BLKC,), jnp.float32),
        pltpu.VMEM((BLKC,), jnp.float32),
        pltpu.VMEM((1, BLKC), jnp.int32),
        pltpu.VMEM((1, BLKC), jnp.int32),
        pltpu.VMEM((BLKC,), jnp.int32),
        pltpu.VMEM((BLKC,), jnp.int32),
        pltpu.VMEM((BLKC, 2 * D), jnp.float32),
        pltpu.VMEM((BLKC, 2 * D), jnp.float32),
        pltpu.VMEM_SHARED((PACC, 2 * D), jnp.float32),
        pltpu.SemaphoreType.DMA,
        pltpu.SemaphoreType.DMA,
    ],
)
def _sc_c(src_hbm, dst_hbm, alpha_hbm, h_hbm, out_hbm,
          srcm, dstm, alpm, alp0, alp1, par0, par1, ld0, ld1,
          si0, si1, rg0, rg1, acc, gs0, gs1):
    cid = lax.axis_index("c")
    sid = lax.axis_index("s")
    lane = lax.iota(jnp.int32, 16)
    wbase = sid * EW_C
    ALP = (alp0, alp1)
    PAR = (par0, par1)
    LDS = (ld0, ld1)
    SIDX = (si0, si1)
    ROWS = (rg0, rg1)
    GS = (gs0, gs1)

    @pl.loop(0, BLKC)
    def _(i):
        @pl.loop(0, 2 * D, step=16)
        def _(q):
            rg0[i, pl.ds(q, 16)] = jnp.zeros((16,), jnp.float32)

    zbase = sid * SLICE_PS
    for t in range(SLICE_PS // FCHUNK):
        pltpu.sync_copy(rg0.at[pl.ds(0, FCHUNK)],
                        acc.at[pl.ds(zbase + t * FCHUNK, FCHUNK)])
    plsc.subcore_barrier()

    def prep(b, s):
        @pl.when((b & (MCH // BLKC - 1)) == 0)
        def _():
            off = wbase + b * BLKC
            pltpu.sync_copy(src_hbm.at[pl.ds(off, MCH)], srcm)
            pltpu.sync_copy(dst_hbm.at[pl.ds(off, MCH)], dstm)
            pltpu.sync_copy(alpha_hbm.at[pl.ds(off, MCH)], alpm)

        moff = (b & (MCH // BLKC - 1)) * BLKC
        eoff = wbase + b * BLKC

        @pl.loop(0, BLKC, step=16)
        def _(j):
            d16 = dstm[pl.ds(moff + j, 16)]
            a = alpm[pl.ds(moff + j, 16)]
            SIDX[s][pl.ds(j, 16)] = srcm[pl.ds(moff + j, 16)]
            ld = d16 - cid * NHALF
            inr = (ld >= 0) & (ld < NHALF)
            ALP[s][pl.ds(j, 16)] = jnp.where(inr, a, 0.0)
            PAR[s][pl.ds(j, 16)] = (ld & 1).astype(jnp.float32)
            LDS[s][0, pl.ds(j, 16)] = jnp.where(
                inr, lax.shift_right_arithmetic(ld, 1),
                (eoff + j + lane) & 2047)

        pltpu.async_copy(h_hbm.at[SIDX[s]], ROWS[s], GS[s])

    def consume(s):
        pltpu.make_async_copy(h_hbm.at[pl.ds(0, BLKC)], ROWS[s], GS[s]).wait()

        @pl.loop(0, BLKC)
        def _(i):
            i16 = jnp.full((16,), i, jnp.int32)
            asp = plsc.load_gather(ALP[s], [i16])
            aR = asp * plsc.load_gather(PAR[s], [i16])
            aL = asp - aR
            for q in range(D // 16):
                v = ROWS[s][i, pl.ds(q * 16, 16)]
                ROWS[s][i, pl.ds(q * 16, 16)] = v * aL
                ROWS[s][i, pl.ds(D + q * 16, 16)] = v * aR

        pltpu.sync_copy(ROWS[s], acc.at[LDS[s].at[0]], add=True)

    prep(0, 0)
    prep(1, 1)

    @pl.loop(0, T_PAIRS - 1)
    def _(t):
        for s in range(2):
            consume(s)
            prep(2 * t + 2 + s, s)

    for s in range(2):
        consume(s)

    plsc.subcore_barrier()

    fbase = sid * SLICE_PS
    for t in range(SLICE_PS // FCHUNK):
        pltpu.sync_copy(acc.at[pl.ds(fbase + t * FCHUNK, FCHUNK)],
                        out_hbm.at[cid, pl.ds(fbase + t * FCHUNK, FCHUNK)])


def _gat_layer(x_args, src_p, dst_p, first):
    if first:
        tab, wt, avs, avd = x_args
        h, a_s, a_d, _, _, c = _tc_h1(tab, wt, avs, avd)
    else:
        xb, hp, srd, b, wt, avs, avd = x_args
        h, a_s, a_d, _, _, c = _tc_h2(xb, hp, srd, b, wt, avs, avd)
    w, parts = _sc_a(src_p, dst_p, a_s.reshape(N), a_d.reshape(N), c)
    rden, srd2 = _tc_r(parts.reshape(32, NR0, NR1), a_s.reshape(NR0, NR1),
                       a_d.reshape(NR0, NR1), c)
    alpha = _sc_b(dst_p, w, rden.reshape(N))
    out_sc = _sc_c(src_p, dst_p, alpha, h).reshape(NC, 2 * PACC, D)
    outcat = jnp.concatenate([out_sc[0, :NHALF], out_sc[1, :NHALF]], axis=0)
    return outcat, h, srd2.reshape(N, 1)


def kernel(edge_index, table, W1, att_src1, att_dst1, b1,
           W2, att_src2, att_dst2, b2):
    padidx = jnp.arange(PAD, dtype=jnp.int32) % 256
    src_p = jnp.concatenate([edge_index[0].astype(jnp.int32), padidx])
    dst_p = jnp.concatenate([edge_index[1].astype(jnp.int32), padidx])

    out1, h1, srd1 = _gat_layer(
        (table, W1.T, att_src1.reshape(1, D), att_dst1.reshape(1, D)),
        src_p, dst_p, first=True)
    out2, h2, srd2 = _gat_layer(
        (out1, h1, srd1, b1.reshape(1, D), W2.T,
         att_src2.reshape(1, D), att_dst2.reshape(1, D)),
        src_p, dst_p, first=False)
    return _tc_fin(out2, h2, srd2, b2.reshape(1, D))


# speedup vs baseline: 24.3691x; 1.1672x over previous
import dataclasses
import functools

import jax
import jax.numpy as jnp
from jax import lax
from jax.experimental import pallas as pl
from jax.experimental.pallas import tpu as pltpu
from jax.experimental.pallas import tpu_sc as plsc

N = 50000
D = 64
E = 800000
SLOPE_ATT = 0.2
SLOPE_ACT = 0.01

NC = 2
NS = 16
EP = 800768
PAD = EP - E
EW_A = EP // 32
BLKA = 1088
NBLKA = EW_A // BLKA
EW_C = EP // 16
BLKC = 64
NBLKC = EW_C // BLKC
NHALF = 25000
PACC = 12544
DUMMY = 12512
SLICE_PS = PACC // NS
FCHUNK = 56

_mesh = plsc.VectorSubcoreMesh(
    core_axis_name="c", subcore_axis_name="s", num_cores=NC, num_subcores=NS)


def _sc_params():
    cp = pltpu.CompilerParams()
    if "needs_layout_passes" in pltpu.CompilerParams.__dataclass_fields__:
        cp = dataclasses.replace(cp, needs_layout_passes=False)
    return cp


def _leaky(x, slope):
    return jnp.maximum(x, slope * x)


R = 2000


def _tc_h_common(x, wt_ref, avs_ref, avd_ref, h_ref, as_ref, ad_ref,
                 ms_ref, md_ref, c_ref):
    i = pl.program_id(0)
    h = jnp.dot(x, wt_ref[...], preferred_element_type=jnp.float32)
    a_s = jnp.sum(h * avs_ref[...], axis=1, keepdims=True)
    a_d = jnp.sum(h * avd_ref[...], axis=1, keepdims=True)
    h_ref[...] = jnp.concatenate([h, jnp.zeros((R, D), jnp.float32)], axis=1)
    as_ref[...] = a_s
    ad_ref[...] = a_d
    bs = jnp.max(a_s).reshape(1, 1)
    bd = jnp.max(a_d).reshape(1, 1)

    @pl.when(i == 0)
    def _():
        ms_ref[...] = bs
        md_ref[...] = bd

    @pl.when(i > 0)
    def _():
        ms_ref[...] = jnp.maximum(ms_ref[...], bs)
        md_ref[...] = jnp.maximum(md_ref[...], bd)

    cc = _leaky(ms_ref[...] + md_ref[...], SLOPE_ATT)
    c_ref[...] = jnp.broadcast_to(cc, (1, 16))


def _tc_h1_body(tab_ref, wt_ref, avs_ref, avd_ref,
                h_ref, as_ref, ad_ref, ms_ref, md_ref, c_ref):
    i = pl.program_id(0)
    rows = lax.broadcasted_iota(jnp.int32, (R, 1), 0) + i * R
    x = jnp.where(rows != 0, tab_ref[...], 0.0)
    _tc_h_common(x, wt_ref, avs_ref, avd_ref, h_ref, as_ref, ad_ref,
                 ms_ref, md_ref, c_ref)


def _tc_h2_body(xb_ref, hp_ref, srd_ref, b_ref, wt_ref, avs_ref, avd_ref,
                h_ref, as_ref, ad_ref, ms_ref, md_ref, c_ref):
    x = _leaky(xb_ref[...] + srd_ref[...] * hp_ref[...][:, :D] + b_ref[...],
               SLOPE_ACT)
    _tc_h_common(x, wt_ref, avs_ref, avd_ref, h_ref, as_ref, ad_ref,
                 ms_ref, md_ref, c_ref)


_h_outs = (
    jax.ShapeDtypeStruct((N, 2 * D), jnp.float32),
    jax.ShapeDtypeStruct((N, 1), jnp.float32),
    jax.ShapeDtypeStruct((N, 1), jnp.float32),
    jax.ShapeDtypeStruct((1, 1), jnp.float32),
    jax.ShapeDtypeStruct((1, 1), jnp.float32),
    jax.ShapeDtypeStruct((1, 16), jnp.float32),
)
_h_out_specs = [
    pl.BlockSpec((R, 2 * D), lambda i: (i, 0)),
    pl.BlockSpec((R, 1), lambda i: (i, 0)),
    pl.BlockSpec((R, 1), lambda i: (i, 0)),
    pl.BlockSpec((1, 1), lambda i: (0, 0)),
    pl.BlockSpec((1, 1), lambda i: (0, 0)),
    pl.BlockSpec((1, 16), lambda i: (0, 0)),
]
_w_specs = [
    pl.BlockSpec((D, D), lambda i: (0, 0)),
    pl.BlockSpec((1, D), lambda i: (0, 0)),
    pl.BlockSpec((1, D), lambda i: (0, 0)),
]

_tc_h1 = pl.pallas_call(
    _tc_h1_body,
    grid=(N // R,),
    in_specs=[pl.BlockSpec((R, D), lambda i: (i, 0))] + _w_specs,
    out_specs=_h_out_specs,
    out_shape=_h_outs,
)

_tc_h2 = pl.pallas_call(
    _tc_h2_body,
    grid=(N // R,),
    in_specs=[
        pl.BlockSpec((R, D), lambda i: (i, 0)),
        pl.BlockSpec((R, 2 * D), lambda i: (i, 0)),
        pl.BlockSpec((R, 1), lambda i: (i, 0)),
        pl.BlockSpec((1, D), lambda i: (0, 0)),
    ] + _w_specs,
    out_specs=_h_out_specs,
    out_shape=_h_outs,
)

BN = 10000


NR0, NR1 = 500, 100


def _tc_r_body(p_ref, as_ref, ad_ref, c_ref, rd_ref, srd_ref):
    cval = c_ref[...][0, 0]
    den = jnp.sum(p_ref[...], axis=0)
    ws = jnp.exp(_leaky(as_ref[...] + ad_ref[...], SLOPE_ATT) - cval)
    r = 1.0 / (den + ws + 1e-16)
    rd_ref[...] = r
    srd_ref[...] = ws * r


_tc_r = pl.pallas_call(
    _tc_r_body,
    grid=(1,),
    in_specs=[
        pl.BlockSpec((32, NR0, NR1), lambda i: (0, 0, 0)),
        pl.BlockSpec((NR0, NR1), lambda i: (0, 0)),
        pl.BlockSpec((NR0, NR1), lambda i: (0, 0)),
        pl.BlockSpec((1, 16), lambda i: (0, 0)),
    ],
    out_specs=[
        pl.BlockSpec((NR0, NR1), lambda i: (0, 0)),
        pl.BlockSpec((NR0, NR1), lambda i: (0, 0)),
    ],
    out_shape=(
        jax.ShapeDtypeStruct((NR0, NR1), jnp.float32),
        jax.ShapeDtypeStruct((NR0, NR1), jnp.float32),
    ),
)


def _tc_fin_body(xb_ref, hp_ref, srd_ref, b_ref, o_ref):
    o_ref[...] = _leaky(
        xb_ref[...] + srd_ref[...] * hp_ref[...][:, :D] + b_ref[...],
        SLOPE_ACT)


_tc_fin = pl.pallas_call(
    _tc_fin_body,
    grid=(N // R,),
    in_specs=[
        pl.BlockSpec((R, D), lambda i: (i, 0)),
        pl.BlockSpec((R, 2 * D), lambda i: (i, 0)),
        pl.BlockSpec((R, 1), lambda i: (i, 0)),
        pl.BlockSpec((1, D), lambda i: (0, 0)),
    ],
    out_specs=pl.BlockSpec((R, D), lambda i: (i, 0)),
    out_shape=jax.ShapeDtypeStruct((N, D), jnp.float32),
)


@functools.partial(
    pl.kernel,
    out_type=(
        jax.ShapeDtypeStruct((EP,), jnp.float32),
        jax.ShapeDtypeStruct((32, 1, N), jnp.float32),
    ),
    mesh=_mesh,
    compiler_params=_sc_params(),
    scratch_types=[
        pltpu.VMEM((N,), jnp.float32),
        pltpu.VMEM((N,), jnp.float32),
        pltpu.VMEM((BLKA,), jnp.int32),
        pltpu.VMEM((BLKA,), jnp.int32),
        pltpu.VMEM((BLKA,), jnp.float32),
        pltpu.VMEM((16,), jnp.float32),
        pltpu.VMEM((16,), jnp.int32),
    ],
)
def _sc_a(src_hbm, dst_hbm, asrc_hbm, adst_hbm, c_hbm,
          w_hbm, part_hbm, A, Bv, sblk, dblk, wblk, cb, tk):
    wid = lax.axis_index("s") * NC + lax.axis_index("c")
    base = wid * EW_A
    pltpu.sync_copy(asrc_hbm, A)
    pltpu.sync_copy(adst_hbm, Bv)
    pltpu.sync_copy(c_hbm.at[0], cb)
    c16 = cb[...]
    lane = lax.iota(jnp.int32, 16)

    @pl.loop(0, NBLKA)
    def _(b):
        off = base + b * BLKA
        pltpu.sync_copy(src_hbm.at[pl.ds(off, BLKA)], sblk)
        pltpu.sync_copy(dst_hbm.at[pl.ds(off, BLKA)], dblk)

        @pl.loop(0, BLKA, step=16)
        def _(j):
            si = sblk[pl.ds(j, 16)]
            di = dblk[pl.ds(j, 16)]
            e = plsc.load_gather(A, [si]) + plsc.load_gather(Bv, [di])
            w = jnp.exp(_leaky(e, SLOPE_ATT) - c16)
            eid = off + j + lane
            wblk[pl.ds(j, 16)] = jnp.where(eid < E, w, 0.0)

        pltpu.sync_copy(wblk, w_hbm.at[pl.ds(off, BLKA)])

    @pl.loop(0, N, step=16)
    def _(i):
        A[pl.ds(i, 16)] = jnp.zeros((16,), jnp.float32)

    @pl.loop(0, NBLKA)
    def _(b):
        off = base + b * BLKA
        pltpu.sync_copy(dst_hbm.at[pl.ds(off, BLKA)], dblk)
        pltpu.sync_copy(w_hbm.at[pl.ds(off, BLKA)], wblk)

        @pl.loop(0, BLKA, step=16)
        def _(j):
            di = dblk[pl.ds(j, 16)]
            w = wblk[pl.ds(j, 16)]
            k, v = plsc.sort_key_val(di, w)
            s = plsc.cumsum(v)
            tk[...] = k
            kn = plsc.load_gather(tk, [jnp.minimum(lane + 1, 15)])
            is_last = (k != kn) | (lane == 15)
            plsc.addupdate_scatter(A, [k], s, mask=is_last)
            plsc.addupdate_scatter(A, [kn], -s, mask=is_last & (lane != 15))

    pltpu.sync_copy(A, part_hbm.at[wid, 0])


@functools.partial(
    pl.kernel,
    out_type=jax.ShapeDtypeStruct((EP,), jnp.float32),
    mesh=_mesh,
    compiler_params=_sc_params(),
    scratch_types=[
        pltpu.VMEM((N,), jnp.float32),
        pltpu.VMEM((BLKA,), jnp.int32),
        pltpu.VMEM((BLKA,), jnp.float32),
        pltpu.VMEM((BLKA,), jnp.float32),
    ],
)
def _sc_b(dst_hbm, w_hbm, rden_hbm, alpha_hbm, rden_t, dblk, wblk, ablk):
    wid = lax.axis_index("s") * NC + lax.axis_index("c")
    base = wid * EW_A
    pltpu.sync_copy(rden_hbm, rden_t)

    @pl.loop(0, NBLKA)
    def _(b):
        off = base + b * BLKA
        pltpu.sync_copy(dst_hbm.at[pl.ds(off, BLKA)], dblk)
        pltpu.sync_copy(w_hbm.at[pl.ds(off, BLKA)], wblk)

        @pl.loop(0, BLKA, step=16)
        def _(j):
            di = dblk[pl.ds(j, 16)]
            ablk[pl.ds(j, 16)] = (
                wblk[pl.ds(j, 16)] * plsc.load_gather(rden_t, [di]))

        pltpu.sync_copy(ablk, alpha_hbm.at[pl.ds(off, BLKA)])


MCH = 1024
T_PAIRS = NBLKC // 2


@functools.partial(
    pl.kernel,
    out_type=jax.ShapeDtypeStruct((NC, PACC, 2 * D), jnp.float32),
    mesh=_mesh,
    compiler_params=_sc_params(),
    scratch_types=[
        pltpu.VMEM((MCH,), jnp.int32),
        pltpu.VMEM((MCH,), jnp.int32),
        pltpu.VMEM((MCH,), jnp.float32),
        pltpu.VMEM((BLKC,), jnp.float32),
        pltpu.VMEM((BLKC,), jnp.float32),
        pltpu.VMEM((BLKC,), jnp.float32),
        pltpu.VMEM((BLKC,), jnp.float32),
        pltpu.VMEM((1, BLKC), jnp.int32),
        pltpu.VMEM((1, BLKC), jnp.int32),
        pltpu.VMEM((BLKC,), jnp.int32),
        pltpu.VMEM((BLKC,), jnp.int32),
        pltpu.VMEM((BLKC, 2 * D), jnp.float32),
        pltpu.VMEM((BLKC, 2 * D), jnp.float32),
        pltpu.VMEM_SHARED((PACC, 2 * D), jnp.float32),
        pltpu.SemaphoreType.DMA,
        pltpu.SemaphoreType.DMA,
    ],
)
def _sc_c(src_hbm, dst_hbm, alpha_hbm, h_hbm, out_hbm,
          srcm, dstm, alpm, alp0, alp1, par0, par1, ld0, ld1,
          si0, si1, rg0, rg1, acc, gs0, gs1):
    cid = lax.axis_index("c")
    sid = lax.axis_index("s")
    lane = lax.iota(jnp.int32, 16)
    wbase = sid * EW_C
    ALP = (alp0, alp1)
    PAR = (par0, par1)
    LDS = (ld0, ld1)
    SIDX = (si0, si1)
    ROWS = (rg0, rg1)
    GS = (gs0, gs1)

    @pl.loop(0, BLKC)
    def _(i):
        @pl.loop(0, 2 * D, step=16)
        def _(q):
            rg0[i, pl.ds(q, 16)] = jnp.zeros((16,), jnp.float32)

    zbase = sid * SLICE_PS
    for t in range(SLICE_PS // FCHUNK):
        pltpu.sync_copy(rg0.at[pl.ds(0, FCHUNK)],
                        acc.at[pl.ds(zbase + t * FCHUNK, FCHUNK)])
    plsc.subcore_barrier()

    def prep(b, s):
        @pl.when((b & (MCH // BLKC - 1)) == 0)
        def _():
            off = wbase + b * BLKC
            pltpu.sync_copy(src_hbm.at[pl.ds(off, MCH)], srcm)
            pltpu.sync_copy(dst_hbm.at[pl.ds(off, MCH)], dstm)
            pltpu.sync_copy(alpha_hbm.at[pl.ds(off, MCH)], alpm)

        moff = (b & (MCH // BLKC - 1)) * BLKC
        eoff = wbase + b * BLKC

        @plsc.parallel_loop(0, BLKC, 16, unroll=2)
        def _(j):
            d16 = dstm[pl.ds(moff + j, 16)]
            a = alpm[pl.ds(moff + j, 16)]
            SIDX[s][pl.ds(j, 16)] = srcm[pl.ds(moff + j, 16)]
            ld = d16 - cid * NHALF
            inr = (ld >= 0) & (ld < NHALF)
            ALP[s][pl.ds(j, 16)] = jnp.where(inr, a, 0.0)
            PAR[s][pl.ds(j, 16)] = (ld & 1).astype(jnp.float32)
            LDS[s][0, pl.ds(j, 16)] = jnp.where(
                inr, lax.shift_right_arithmetic(ld, 1),
                (eoff + j + lane) & 2047)

        pltpu.async_copy(h_hbm.at[SIDX[s]], ROWS[s], GS[s])

    def consume(s):
        pltpu.make_async_copy(h_hbm.at[pl.ds(0, BLKC)], ROWS[s], GS[s]).wait()

        @plsc.parallel_loop(0, BLKC, 1, unroll=4)
        def _(i):
            i16 = jnp.full((16,), i, jnp.int32)
            asp = plsc.load_gather(ALP[s], [i16])
            aR = asp * plsc.load_gather(PAR[s], [i16])
            aL = asp - aR
            for q in range(D // 16):
                v = ROWS[s][i, pl.ds(q * 16, 16)]
                ROWS[s][i, pl.ds(q * 16, 16)] = v * aL
                ROWS[s][i, pl.ds(D + q * 16, 16)] = v * aR

        pltpu.sync_copy(ROWS[s], acc.at[LDS[s].at[0]], add=True)

    prep(0, 0)
    prep(1, 1)

    @pl.loop(0, T_PAIRS - 1)
    def _(t):
        for s in range(2):
            consume(s)
            prep(2 * t + 2 + s, s)

    for s in range(2):
        consume(s)

    plsc.subcore_barrier()

    fbase = sid * SLICE_PS
    for t in range(SLICE_PS // FCHUNK):
        pltpu.sync_copy(acc.at[pl.ds(fbase + t * FCHUNK, FCHUNK)],
                        out_hbm.at[cid, pl.ds(fbase + t * FCHUNK, FCHUNK)])


def _gat_layer(x_args, src_p, dst_p, first):
    if first:
        tab, wt, avs, avd = x_args
        h, a_s, a_d, _, _, c = _tc_h1(tab, wt, avs, avd)
    else:
        xb, hp, srd, b, wt, avs, avd = x_args
        h, a_s, a_d, _, _, c = _tc_h2(xb, hp, srd, b, wt, avs, avd)
    w, parts = _sc_a(src_p, dst_p, a_s.reshape(N), a_d.reshape(N), c)
    rden, srd2 = _tc_r(parts.reshape(32, NR0, NR1), a_s.reshape(NR0, NR1),
                       a_d.reshape(NR0, NR1), c)
    alpha = _sc_b(dst_p, w, rden.reshape(N))
    out_sc = _sc_c(src_p, dst_p, alpha, h).reshape(NC, 2 * PACC, D)
    outcat = jnp.concatenate([out_sc[0, :NHALF], out_sc[1, :NHALF]], axis=0)
    return outcat, h, srd2.reshape(N, 1)


def kernel(edge_index, table, W1, att_src1, att_dst1, b1,
           W2, att_src2, att_dst2, b2):
    padidx = jnp.arange(PAD, dtype=jnp.int32) % 256
    src_p = jnp.concatenate([edge_index[0].astype(jnp.int32), padidx])
    dst_p = jnp.concatenate([edge_index[1].astype(jnp.int32), padidx])

    out1, h1, srd1 = _gat_layer(
        (table, W1.T, att_src1.reshape(1, D), att_dst1.reshape(1, D)),
        src_p, dst_p, first=True)
    out2, h2, srd2 = _gat_layer(
        (out1, h1, srd1, b1.reshape(1, D), W2.T,
         att_src2.reshape(1, D), att_dst2.reshape(1, D)),
        src_p, dst_p, first=False)
    return _tc_fin(out2, h2, srd2, b2.reshape(1, D))


# speedup vs baseline: 25.1495x; 1.0320x over previous
import dataclasses
import functools

import jax
import jax.numpy as jnp
from jax import lax
from jax.experimental import pallas as pl
from jax.experimental.pallas import tpu as pltpu
from jax.experimental.pallas import tpu_sc as plsc

N = 50000
D = 64
E = 800000
SLOPE_ATT = 0.2
SLOPE_ACT = 0.01

NC = 2
NS = 16
EP = 800768
PAD = EP - E
EW_A = EP // 32
BLKA = 1088
NBLKA = EW_A // BLKA
EW_C = EP // 16
BLKC = 64
NBLKC = EW_C // BLKC
NHALF = 25000
PACC = 12544
DUMMY = 12512
SLICE_PS = PACC // NS
FCHUNK = 56

_mesh = plsc.VectorSubcoreMesh(
    core_axis_name="c", subcore_axis_name="s", num_cores=NC, num_subcores=NS)


def _sc_params():
    cp = pltpu.CompilerParams()
    if "needs_layout_passes" in pltpu.CompilerParams.__dataclass_fields__:
        cp = dataclasses.replace(cp, needs_layout_passes=False)
    return cp


def _leaky(x, slope):
    return jnp.maximum(x, slope * x)


R = 2000


def _tc_h_common(x, wt_ref, avs_ref, avd_ref, h_ref, as_ref, ad_ref,
                 ms_ref, md_ref, c_ref):
    i = pl.program_id(0)
    h = jnp.dot(x, wt_ref[...], preferred_element_type=jnp.float32)
    a_s = jnp.sum(h * avs_ref[...], axis=1, keepdims=True)
    a_d = jnp.sum(h * avd_ref[...], axis=1, keepdims=True)
    h_ref[...] = jnp.concatenate([h, jnp.zeros((R, D), jnp.float32)], axis=1)
    as_ref[...] = a_s
    ad_ref[...] = a_d
    bs = jnp.max(a_s).reshape(1, 1)
    bd = jnp.max(a_d).reshape(1, 1)

    @pl.when(i == 0)
    def _():
        ms_ref[...] = bs
        md_ref[...] = bd

    @pl.when(i > 0)
    def _():
        ms_ref[...] = jnp.maximum(ms_ref[...], bs)
        md_ref[...] = jnp.maximum(md_ref[...], bd)

    cc = _leaky(ms_ref[...] + md_ref[...], SLOPE_ATT)
    c_ref[...] = jnp.broadcast_to(cc, (1, 16))


def _tc_h1_body(tab_ref, wt_ref, avs_ref, avd_ref,
                h_ref, as_ref, ad_ref, ms_ref, md_ref, c_ref):
    i = pl.program_id(0)
    rows = lax.broadcasted_iota(jnp.int32, (R, 1), 0) + i * R
    x = jnp.where(rows != 0, tab_ref[...], 0.0)
    _tc_h_common(x, wt_ref, avs_ref, avd_ref, h_ref, as_ref, ad_ref,
                 ms_ref, md_ref, c_ref)


def _tc_h2_body(xb_ref, hp_ref, srd_ref, b_ref, wt_ref, avs_ref, avd_ref,
                h_ref, as_ref, ad_ref, ms_ref, md_ref, c_ref):
    x = _leaky(xb_ref[...] + srd_ref[...] * hp_ref[...][:, :D] + b_ref[...],
               SLOPE_ACT)
    _tc_h_common(x, wt_ref, avs_ref, avd_ref, h_ref, as_ref, ad_ref,
                 ms_ref, md_ref, c_ref)


_h_outs = (
    jax.ShapeDtypeStruct((N, 2 * D), jnp.float32),
    jax.ShapeDtypeStruct((N, 1), jnp.float32),
    jax.ShapeDtypeStruct((N, 1), jnp.float32),
    jax.ShapeDtypeStruct((1, 1), jnp.float32),
    jax.ShapeDtypeStruct((1, 1), jnp.float32),
    jax.ShapeDtypeStruct((1, 16), jnp.float32),
)
_h_out_specs = [
    pl.BlockSpec((R, 2 * D), lambda i: (i, 0)),
    pl.BlockSpec((R, 1), lambda i: (i, 0)),
    pl.BlockSpec((R, 1), lambda i: (i, 0)),
    pl.BlockSpec((1, 1), lambda i: (0, 0)),
    pl.BlockSpec((1, 1), lambda i: (0, 0)),
    pl.BlockSpec((1, 16), lambda i: (0, 0)),
]
_w_specs = [
    pl.BlockSpec((D, D), lambda i: (0, 0)),
    pl.BlockSpec((1, D), lambda i: (0, 0)),
    pl.BlockSpec((1, D), lambda i: (0, 0)),
]

_tc_h1 = pl.pallas_call(
    _tc_h1_body,
    grid=(N // R,),
    in_specs=[pl.BlockSpec((R, D), lambda i: (i, 0))] + _w_specs,
    out_specs=_h_out_specs,
    out_shape=_h_outs,
)

_tc_h2 = pl.pallas_call(
    _tc_h2_body,
    grid=(N // R,),
    in_specs=[
        pl.BlockSpec((R, D), lambda i: (i, 0)),
        pl.BlockSpec((R, 2 * D), lambda i: (i, 0)),
        pl.BlockSpec((R, 1), lambda i: (i, 0)),
        pl.BlockSpec((1, D), lambda i: (0, 0)),
    ] + _w_specs,
    out_specs=_h_out_specs,
    out_shape=_h_outs,
)

BN = 10000


NR0, NR1 = 500, 100


def _tc_r_body(p_ref, as_ref, ad_ref, c_ref, rd_ref, srd_ref):
    cval = c_ref[...][0, 0]
    den = jnp.sum(p_ref[...], axis=0)
    ws = jnp.exp(_leaky(as_ref[...] + ad_ref[...], SLOPE_ATT) - cval)
    r = 1.0 / (den + ws + 1e-16)
    rd_ref[...] = r
    srd_ref[...] = ws * r


_tc_r = pl.pallas_call(
    _tc_r_body,
    grid=(1,),
    in_specs=[
        pl.BlockSpec((32, NR0, NR1), lambda i: (0, 0, 0)),
        pl.BlockSpec((NR0, NR1), lambda i: (0, 0)),
        pl.BlockSpec((NR0, NR1), lambda i: (0, 0)),
        pl.BlockSpec((1, 16), lambda i: (0, 0)),
    ],
    out_specs=[
        pl.BlockSpec((NR0, NR1), lambda i: (0, 0)),
        pl.BlockSpec((NR0, NR1), lambda i: (0, 0)),
    ],
    out_shape=(
        jax.ShapeDtypeStruct((NR0, NR1), jnp.float32),
        jax.ShapeDtypeStruct((NR0, NR1), jnp.float32),
    ),
)


def _tc_fin_body(xb_ref, hp_ref, srd_ref, b_ref, o_ref):
    o_ref[...] = _leaky(
        xb_ref[...] + srd_ref[...] * hp_ref[...][:, :D] + b_ref[...],
        SLOPE_ACT)


_tc_fin = pl.pallas_call(
    _tc_fin_body,
    grid=(N // R,),
    in_specs=[
        pl.BlockSpec((R, D), lambda i: (i, 0)),
        pl.BlockSpec((R, 2 * D), lambda i: (i, 0)),
        pl.BlockSpec((R, 1), lambda i: (i, 0)),
        pl.BlockSpec((1, D), lambda i: (0, 0)),
    ],
    out_specs=pl.BlockSpec((R, D), lambda i: (i, 0)),
    out_shape=jax.ShapeDtypeStruct((N, D), jnp.float32),
)


@functools.partial(
    pl.kernel,
    out_type=(
        jax.ShapeDtypeStruct((EP,), jnp.float32),
        jax.ShapeDtypeStruct((32, 1, N), jnp.float32),
    ),
    mesh=_mesh,
    compiler_params=_sc_params(),
    scratch_types=[
        pltpu.VMEM((N,), jnp.float32),
        pltpu.VMEM((N,), jnp.float32),
        pltpu.VMEM((BLKA,), jnp.int32),
        pltpu.VMEM((BLKA,), jnp.int32),
        pltpu.VMEM((BLKA,), jnp.float32),
        pltpu.VMEM((16,), jnp.float32),
        pltpu.VMEM((16,), jnp.int32),
    ],
)
def _sc_a(src_hbm, dst_hbm, asrc_hbm, adst_hbm, c_hbm,
          w_hbm, part_hbm, A, Bv, sblk, dblk, wblk, cb, tk):
    wid = lax.axis_index("s") * NC + lax.axis_index("c")
    base = wid * EW_A
    pltpu.sync_copy(asrc_hbm, A)
    pltpu.sync_copy(adst_hbm, Bv)
    pltpu.sync_copy(c_hbm.at[0], cb)
    c16 = cb[...]
    lane = lax.iota(jnp.int32, 16)

    @pl.loop(0, NBLKA)
    def _(b):
        off = base + b * BLKA
        pltpu.sync_copy(src_hbm.at[pl.ds(off, BLKA)], sblk)
        pltpu.sync_copy(dst_hbm.at[pl.ds(off, BLKA)], dblk)

        @plsc.parallel_loop(0, BLKA, 16, unroll=4)
        def _(j):
            si = sblk[pl.ds(j, 16)]
            di = dblk[pl.ds(j, 16)]
            e = plsc.load_gather(A, [si]) + plsc.load_gather(Bv, [di])
            w = jnp.exp(_leaky(e, SLOPE_ATT) - c16)
            eid = off + j + lane
            wblk[pl.ds(j, 16)] = jnp.where(eid < E, w, 0.0)

        pltpu.sync_copy(wblk, w_hbm.at[pl.ds(off, BLKA)])

    @plsc.parallel_loop(0, N, 16, unroll=4)
    def _(i):
        A[pl.ds(i, 16)] = jnp.zeros((16,), jnp.float32)

    @pl.loop(0, NBLKA)
    def _(b):
        off = base + b * BLKA
        pltpu.sync_copy(dst_hbm.at[pl.ds(off, BLKA)], dblk)
        pltpu.sync_copy(w_hbm.at[pl.ds(off, BLKA)], wblk)

        @pl.loop(0, BLKA, step=16)
        def _(j):
            di = dblk[pl.ds(j, 16)]
            w = wblk[pl.ds(j, 16)]
            k, v = plsc.sort_key_val(di, w)
            s = plsc.cumsum(v)
            tk[...] = k
            kn = plsc.load_gather(tk, [jnp.minimum(lane + 1, 15)])
            is_last = (k != kn) | (lane == 15)
            plsc.addupdate_scatter(A, [k], s, mask=is_last)
            plsc.addupdate_scatter(A, [kn], -s, mask=is_last & (lane != 15))

    pltpu.sync_copy(A, part_hbm.at[wid, 0])


@functools.partial(
    pl.kernel,
    out_type=jax.ShapeDtypeStruct((EP,), jnp.float32),
    mesh=_mesh,
    compiler_params=_sc_params(),
    scratch_types=[
        pltpu.VMEM((N,), jnp.float32),
        pltpu.VMEM((BLKA,), jnp.int32),
        pltpu.VMEM((BLKA,), jnp.float32),
        pltpu.VMEM((BLKA,), jnp.float32),
    ],
)
def _sc_b(dst_hbm, w_hbm, rden_hbm, alpha_hbm, rden_t, dblk, wblk, ablk):
    wid = lax.axis_index("s") * NC + lax.axis_index("c")
    base = wid * EW_A
    pltpu.sync_copy(rden_hbm, rden_t)

    @pl.loop(0, NBLKA)
    def _(b):
        off = base + b * BLKA
        pltpu.sync_copy(dst_hbm.at[pl.ds(off, BLKA)], dblk)
        pltpu.sync_copy(w_hbm.at[pl.ds(off, BLKA)], wblk)

        @plsc.parallel_loop(0, BLKA, 16, unroll=4)
        def _(j):
            di = dblk[pl.ds(j, 16)]
            ablk[pl.ds(j, 16)] = (
                wblk[pl.ds(j, 16)] * plsc.load_gather(rden_t, [di]))

        pltpu.sync_copy(ablk, alpha_hbm.at[pl.ds(off, BLKA)])


MCH = 1024
T_PAIRS = NBLKC // 2


@functools.partial(
    pl.kernel,
    out_type=jax.ShapeDtypeStruct((NC, PACC, 2 * D), jnp.float32),
    mesh=_mesh,
    compiler_params=_sc_params(),
    scratch_types=[
        pltpu.VMEM((MCH,), jnp.int32),
        pltpu.VMEM((MCH,), jnp.int32),
        pltpu.VMEM((MCH,), jnp.float32),
        pltpu.VMEM((BLKC,), jnp.float32),
        pltpu.VMEM((BLKC,), jnp.float32),
        pltpu.VMEM((BLKC,), jnp.float32),
        pltpu.VMEM((BLKC,), jnp.float32),
        pltpu.VMEM((1, BLKC), jnp.int32),
        pltpu.VMEM((1, BLKC), jnp.int32),
        pltpu.VMEM((BLKC,), jnp.int32),
        pltpu.VMEM((BLKC,), jnp.int32),
        pltpu.VMEM((BLKC, 2 * D), jnp.float32),
        pltpu.VMEM((BLKC, 2 * D), jnp.float32),
        pltpu.VMEM_SHARED((PACC, 2 * D), jnp.float32),
        pltpu.SemaphoreType.DMA,
        pltpu.SemaphoreType.DMA,
    ],
)
def _sc_c(src_hbm, dst_hbm, alpha_hbm, h_hbm, out_hbm,
          srcm, dstm, alpm, alp0, alp1, par0, par1, ld0, ld1,
          si0, si1, rg0, rg1, acc, gs0, gs1):
    cid = lax.axis_index("c")
    sid = lax.axis_index("s")
    lane = lax.iota(jnp.int32, 16)
    wbase = sid * EW_C
    ALP = (alp0, alp1)
    PAR = (par0, par1)
    LDS = (ld0, ld1)
    SIDX = (si0, si1)
    ROWS = (rg0, rg1)
    GS = (gs0, gs1)

    @pl.loop(0, BLKC)
    def _(i):
        @pl.loop(0, 2 * D, step=16)
        def _(q):
            rg0[i, pl.ds(q, 16)] = jnp.zeros((16,), jnp.float32)

    zbase = sid * SLICE_PS
    for t in range(SLICE_PS // FCHUNK):
        pltpu.sync_copy(rg0.at[pl.ds(0, FCHUNK)],
                        acc.at[pl.ds(zbase + t * FCHUNK, FCHUNK)])
    plsc.subcore_barrier()

    def prep(b, s):
        @pl.when((b & (MCH // BLKC - 1)) == 0)
        def _():
            off = wbase + b * BLKC
            pltpu.sync_copy(src_hbm.at[pl.ds(off, MCH)], srcm)
            pltpu.sync_copy(dst_hbm.at[pl.ds(off, MCH)], dstm)
            pltpu.sync_copy(alpha_hbm.at[pl.ds(off, MCH)], alpm)

        moff = (b & (MCH // BLKC - 1)) * BLKC
        eoff = wbase + b * BLKC

        @plsc.parallel_loop(0, BLKC, 16, unroll=2)
        def _(j):
            d16 = dstm[pl.ds(moff + j, 16)]
            a = alpm[pl.ds(moff + j, 16)]
            SIDX[s][pl.ds(j, 16)] = srcm[pl.ds(moff + j, 16)]
            ld = d16 - cid * NHALF
            inr = (ld >= 0) & (ld < NHALF)
            ALP[s][pl.ds(j, 16)] = jnp.where(inr, a, 0.0)
            PAR[s][pl.ds(j, 16)] = (ld & 1).astype(jnp.float32)
            LDS[s][0, pl.ds(j, 16)] = jnp.where(
                inr, lax.shift_right_arithmetic(ld, 1),
                (eoff + j + lane) & 2047)

        pltpu.async_copy(h_hbm.at[SIDX[s]], ROWS[s], GS[s])

    def consume(s):
        pltpu.make_async_copy(h_hbm.at[pl.ds(0, BLKC)], ROWS[s], GS[s]).wait()

        @plsc.parallel_loop(0, BLKC, 1, unroll=4)
        def _(i):
            i16 = jnp.full((16,), i, jnp.int32)
            asp = plsc.load_gather(ALP[s], [i16])
            aR = asp * plsc.load_gather(PAR[s], [i16])
            aL = asp - aR
            for q in range(D // 16):
                v = ROWS[s][i, pl.ds(q * 16, 16)]
                ROWS[s][i, pl.ds(q * 16, 16)] = v * aL
                ROWS[s][i, pl.ds(D + q * 16, 16)] = v * aR

        pltpu.sync_copy(ROWS[s], acc.at[LDS[s].at[0]], add=True)

    prep(0, 0)
    prep(1, 1)

    @pl.loop(0, T_PAIRS - 1)
    def _(t):
        for s in range(2):
            consume(s)
            prep(2 * t + 2 + s, s)

    for s in range(2):
        consume(s)

    plsc.subcore_barrier()

    fbase = sid * SLICE_PS
    for t in range(SLICE_PS // FCHUNK):
        pltpu.sync_copy(acc.at[pl.ds(fbase + t * FCHUNK, FCHUNK)],
                        out_hbm.at[cid, pl.ds(fbase + t * FCHUNK, FCHUNK)])


def _gat_layer(x_args, src_p, dst_p, first):
    if first:
        tab, wt, avs, avd = x_args
        h, a_s, a_d, _, _, c = _tc_h1(tab, wt, avs, avd)
    else:
        xb, hp, srd, b, wt, avs, avd = x_args
        h, a_s, a_d, _, _, c = _tc_h2(xb, hp, srd, b, wt, avs, avd)
    w, parts = _sc_a(src_p, dst_p, a_s.reshape(N), a_d.reshape(N), c)
    rden, srd2 = _tc_r(parts.reshape(32, NR0, NR1), a_s.reshape(NR0, NR1),
                       a_d.reshape(NR0, NR1), c)
    alpha = _sc_b(dst_p, w, rden.reshape(N))
    out_sc = _sc_c(src_p, dst_p, alpha, h).reshape(NC, 2 * PACC, D)
    outcat = jnp.concatenate([out_sc[0, :NHALF], out_sc[1, :NHALF]], axis=0)
    return outcat, h, srd2.reshape(N, 1)


def kernel(edge_index, table, W1, att_src1, att_dst1, b1,
           W2, att_src2, att_dst2, b2):
    padidx = jnp.arange(PAD, dtype=jnp.int32) % 256
    src_p = jnp.concatenate([edge_index[0].astype(jnp.int32), padidx])
    dst_p = jnp.concatenate([edge_index[1].astype(jnp.int32), padidx])

    out1, h1, srd1 = _gat_layer(
        (table, W1.T, att_src1.reshape(1, D), att_dst1.reshape(1, D)),
        src_p, dst_p, first=True)
    out2, h2, srd2 = _gat_layer(
        (out1, h1, srd1, b1.reshape(1, D), W2.T,
         att_src2.reshape(1, D), att_dst2.reshape(1, D)),
        src_p, dst_p, first=False)
    return _tc_fin(out2, h2, srd2, b2.reshape(1, D))
